# baseline (speedup 1.0000x reference)
import jax
import jax.numpy as jnp
from jax.experimental import pallas as pl
from jax.experimental.pallas import tpu as pltpu

IN_DIM = 784
INTER = 512
SD = 16
NS = 256
HID = SD * NS
K_NEURONS = 256
K_STRIPES = 32
BETA = 1.5
GAMMA = 0

BLK = 256


def _count_ge_t(mask_vals, tail_dtype=jnp.int32):
    m = mask_vals
    w = m.shape[0]
    while w > 16:
        w //= 2
        m = m[:w] + m[w:2 * w]
    return jnp.sum(m.astype(tail_dtype), axis=0, keepdims=True)


def _kth_thresh16_t(bits, k):
    one = jnp.int16(1)
    zero = jnp.int16(0)
    k16 = ((bits >> 15) - 32768).astype(jnp.int16)
    lo15 = (bits & 0x7FFF).astype(jnp.int16)
    cols = bits.shape[1]

    U = jnp.zeros((1, cols), jnp.int32)
    for b in range(15, -1, -1):
        cand = U | jnp.int32(1 << b)
        cand16 = (cand - 32768).astype(jnp.int16)
        cnt = _count_ge_t(jnp.where(k16 >= cand16, one, zero))
        U = jnp.where(cnt >= k, cand, U)

    U16 = (U - 32768).astype(jnp.int16)
    eq = k16 == U16
    n_eq = _count_ge_t(jnp.where(eq, one, zero))
    cnt_geU = _count_ge_t(jnp.where(k16 >= U16, one, zero))
    k2 = k - (cnt_geU - n_eq)

    lo15m = jnp.where(eq, lo15, jnp.int16(-1))
    V = jnp.zeros((1, cols), jnp.int32)
    for b in range(14, -1, -1):
        cand = V | jnp.int32(1 << b)
        cand16 = cand.astype(jnp.int16)
        cnt = _count_ge_t(jnp.where(lo15m >= cand16, one, zero))
        V = jnp.where(cnt >= k2, cand, V)
    return (U << 15) | V


def _kth_thresh32_t(bits, k):
    cols = bits.shape[1]
    T = jnp.zeros((1, cols), jnp.int32)
    for b in range(30, -1, -1):
        cand = T | jnp.int32(1 << b)
        cnt = jnp.sum((bits >= cand).astype(jnp.int32), axis=0, keepdims=True)
        T = jnp.where(cnt >= k, cand, T)
    return T


def _fused(x_ref, w1_ref, b1_ref, w2_ref, b2_ref, w3_ref, b3_ref, w4_ref,
           b4_ref, boosts_ref, out_ref):
    x = x_ref[...]
    h1 = jnp.maximum(
        jnp.dot(x, w1_ref[...], preferred_element_type=jnp.float32) + b1_ref[...], 0.0)
    h2 = jnp.maximum(
        jnp.dot(h1, w2_ref[...], preferred_element_type=jnp.float32) + b2_ref[...], 0.0)

    h2t = h2.T
    boosted = h2t * boosts_ref[...]
    bits = jax.lax.bitcast_convert_type(boosted, jnp.int32)
    T = _kth_thresh16_t(bits, K_NEURONS)
    hmt = jnp.where(bits >= T, h2t, 0.0)

    ssum = jnp.sum(hmt.reshape(NS, SD, BLK), axis=1)
    sbits = jax.lax.bitcast_convert_type(ssum, jnp.int32)
    T2 = _kth_thresh32_t(sbits, K_STRIPES)
    smask = (sbits >= T2).astype(jnp.float32)
    sexp = jnp.broadcast_to(smask[:, None, :], (NS, SD, BLK)).reshape(HID, BLK)
    hft = hmt * sexp

    d = jnp.maximum(
        jnp.dot(w3_ref[...], hft, preferred_element_type=jnp.float32) + b3_ref[...], 0.0)
    outt = jnp.maximum(
        jnp.dot(w4_ref[...], d, preferred_element_type=jnp.float32) + b4_ref[...], 0.0)
    out_ref[...] = outt.T


def kernel(x, W1, b1, W2, b2, W3, b3, W4, b4, boosted_scores):
    B = x.shape[0]
    grid = B // BLK
    boosts = jnp.exp(BETA * (GAMMA - boosted_scores)).reshape(HID, 1)

    full = lambda shape: pl.BlockSpec(shape, lambda i: (0, 0))
    out = pl.pallas_call(
        _fused,
        grid=(grid,),
        in_specs=[
            pl.BlockSpec((BLK, IN_DIM), lambda i: (i, 0)),
            full((IN_DIM, INTER)),
            full((1, INTER)),
            full((INTER, HID)),
            full((1, HID)),
            full((INTER, HID)),
            full((INTER, 1)),
            full((IN_DIM, INTER)),
            full((IN_DIM, 1)),
            full((HID, 1)),
        ],
        out_specs=pl.BlockSpec((BLK, IN_DIM), lambda i: (i, 0)),
        out_shape=jax.ShapeDtypeStruct((B, IN_DIM), jnp.float32),
        compiler_params=pltpu.CompilerParams(
            dimension_semantics=("arbitrary",),
        ),
    )(x, W1.T, b1.reshape(1, INTER), W2.T, b2.reshape(1, HID),
      W3, b3.reshape(INTER, 1), W4, b4.reshape(IN_DIM, 1),
      boosts)
    return out
